# speedup vs baseline: 2.9761x; 2.9761x over previous
import functools

import jax
import jax.numpy as jnp
from jax import lax
from jax.experimental import pallas as pl
from jax.experimental.pallas import tpu as pltpu
from jax.experimental.pallas import tpu_sc as plsc

NUM_EMB = 100000
D = 128
B_TOTAL = 4096 * 50
NC = 2
NS = 16
NW = NC * NS
B_PER_W = B_TOTAL // NW
C = 128
NCH = B_PER_W // C


@functools.partial(
    pl.kernel,
    mesh=plsc.VectorSubcoreMesh(core_axis_name="c", subcore_axis_name="s"),
    out_type=jax.ShapeDtypeStruct((NW, NCH, C, D), jnp.float32),
    scratch_types=[
        pltpu.VMEM((NCH, C), jnp.int32),
        pltpu.VMEM((C, D), jnp.float32),
        pltpu.SemaphoreType.DMA,
    ],
)
def _gather_kernel(idx_hbm, table_hbm, out_hbm, idx_v, rows_v, sem):
    wid = lax.axis_index("s") * NC + lax.axis_index("c")
    pltpu.sync_copy(idx_hbm.at[wid], idx_v)

    def chunk(j, carry):
        pltpu.async_copy(table_hbm.at[idx_v.at[j]], rows_v, sem).wait()
        pltpu.sync_copy(rows_v, out_hbm.at[wid, j])
        return carry

    lax.fori_loop(0, NCH, chunk, 0)


def kernel(embedding_property, table):
    idx = embedding_property.astype(jnp.int32).reshape(NW, NCH, C)
    out = _gather_kernel(idx, table)
    return out.reshape(embedding_property.shape + (D,))


# speedup vs baseline: 3.3180x; 1.1149x over previous
import functools

import jax
import jax.numpy as jnp
from jax import lax
from jax.experimental import pallas as pl
from jax.experimental.pallas import tpu as pltpu
from jax.experimental.pallas import tpu_sc as plsc

NUM_EMB = 100000
D = 128
B_TOTAL = 4096 * 50
NC = 2
NS = 16
NW = NC * NS
B_PER_W = B_TOTAL // NW
C = 128
NCH = B_PER_W // C
NBUF = 5
NG = NCH // NBUF


@functools.partial(
    pl.kernel,
    mesh=plsc.VectorSubcoreMesh(core_axis_name="c", subcore_axis_name="s"),
    out_type=jax.ShapeDtypeStruct((NW, NCH, C, D), jnp.float32),
    scratch_types=[
        pltpu.VMEM((NCH, C), jnp.int32),
        pltpu.VMEM((NBUF, C, D), jnp.float32),
        [pltpu.SemaphoreType.DMA] * NBUF,
        [pltpu.SemaphoreType.DMA] * NBUF,
    ],
)
def _gather_kernel(idx_hbm, table_hbm, out_hbm, idx_v, rows_v, gsem, wsem):
    wid = lax.axis_index("s") * NC + lax.axis_index("c")
    pltpu.sync_copy(idx_hbm.at[wid], idx_v)

    for b in range(NBUF):
        pltpu.async_copy(table_hbm.at[idx_v.at[b]], rows_v.at[b], gsem[b])

    def group(g, carry):
        for b in range(NBUF):
            j = g * NBUF + b
            pltpu.make_async_copy(
                table_hbm.at[idx_v.at[j]], rows_v.at[b], gsem[b]
            ).wait()
            pltpu.async_copy(rows_v.at[b], out_hbm.at[wid, j], wsem[b])

        @pl.when(g + 1 < NG)
        def _():
            for b in range(NBUF):
                j = g * NBUF + b
                jn = j + NBUF
                pltpu.make_async_copy(
                    rows_v.at[b], out_hbm.at[wid, j], wsem[b]
                ).wait()
                pltpu.async_copy(
                    table_hbm.at[idx_v.at[jn]], rows_v.at[b], gsem[b]
                )

        return carry

    lax.fori_loop(0, NG, group, 0)

    for b in range(NBUF):
        j = (NG - 1) * NBUF + b
        pltpu.make_async_copy(rows_v.at[b], out_hbm.at[wid, j], wsem[b]).wait()


def kernel(embedding_property, table):
    idx = embedding_property.astype(jnp.int32).reshape(NW, NCH, C)
    out = _gather_kernel(idx, table)
    return out.reshape(embedding_property.shape + (D,))


# speedup vs baseline: 10.1987x; 3.0738x over previous
import functools

import jax
import jax.numpy as jnp
from jax import lax
from jax.experimental import pallas as pl
from jax.experimental.pallas import tpu as pltpu
from jax.experimental.pallas import tpu_sc as plsc

NUM_EMB = 100000
D = 128
B_TOTAL = 4096 * 50
NC = 2
NS = 16
NW = NC * NS
B_PER_W = B_TOTAL // NW
C = 128
NCH = B_PER_W // C
NBUF = 5
NG = NCH // NBUF


@functools.partial(
    pl.kernel,
    mesh=plsc.VectorSubcoreMesh(core_axis_name="c", subcore_axis_name="s"),
    out_type=jax.ShapeDtypeStruct((NW * NCH, C, D), jnp.float32),
    scratch_types=[
        pltpu.VMEM((NCH, C), jnp.int32),
        pltpu.VMEM((NBUF, C, D), jnp.float32),
        [pltpu.SemaphoreType.DMA] * NBUF,
        [pltpu.SemaphoreType.DMA] * NBUF,
    ],
)
def _gather_kernel(idx_hbm, table_hbm, out_hbm, idx_v, rows_v, gsem, wsem):
    wid = lax.axis_index("s") * NC + lax.axis_index("c")
    pltpu.sync_copy(idx_hbm.at[wid], idx_v)

    for b in range(NBUF):
        pltpu.async_copy(table_hbm.at[idx_v.at[b]], rows_v.at[b], gsem[b])

    def group(g, carry):
        for b in range(NBUF):
            j = g * NBUF + b
            pltpu.make_async_copy(
                table_hbm.at[idx_v.at[j]], rows_v.at[b], gsem[b]
            ).wait()
            pltpu.async_copy(rows_v.at[b], out_hbm.at[wid * NCH + j], wsem[b])

        @pl.when(g + 1 < NG)
        def _():
            for b in range(NBUF):
                j = g * NBUF + b
                jn = j + NBUF
                pltpu.make_async_copy(
                    rows_v.at[b], out_hbm.at[wid * NCH + j], wsem[b]
                ).wait()
                pltpu.async_copy(
                    table_hbm.at[idx_v.at[jn]], rows_v.at[b], gsem[b]
                )

        return carry

    lax.fori_loop(0, NG, group, 0)

    for b in range(NBUF):
        j = (NG - 1) * NBUF + b
        pltpu.make_async_copy(rows_v.at[b], out_hbm.at[wid * NCH + j], wsem[b]).wait()


def kernel(embedding_property, table):
    nb, ns = embedding_property.shape
    idx = embedding_property.astype(jnp.int32).T.reshape(NW, NCH, C)
    out = _gather_kernel(idx, table)
    return out.reshape(ns, nb, D).transpose(1, 0, 2)
